# baseline (device time: 133890 ns/iter reference)
import jax
import jax.numpy as jnp
from jax import lax
from jax.experimental import pallas as pl
from jax.experimental.pallas import tpu as pltpu

N_Z = 4
S = 1024
D = 2048
DC = 128
H = 16
DH = 128
DR = 32
HB = H // N_Z
DHB = HB * DH
DRB = HB * DR
SCALE = (DH + DR) ** -0.5

T_C, T_UK, T_UV, T_O = 0, 1, 2, 3


def _body(x_ref, wdkv_ref, wuk_ref, wuv_ref, wq_ref, wqr_ref, wkr_ref,
          wo_ref, out_ref, gc, guk, guv, qs, qrs, ks, vs, o_slots,
          send_sems, recv_sems, o_send_sems, o_recv_sems):
    f32 = jnp.float32
    bf16 = jnp.bfloat16
    my_x = lax.axis_index("x")
    my_y = lax.axis_index("y")
    my_z = lax.axis_index("z")

    xv = x_ref[...]
    col0 = pl.multiple_of(my_z * DHB, DHB)

    c_loc = jnp.dot(xv, wdkv_ref[...], preferred_element_type=f32)
    gc[my_z] = c_loc.astype(bf16)
    guk[my_z] = wuk_ref[:, pl.ds(col0, DHB)]
    guv[my_z] = wuv_ref[:, pl.ds(col0, DHB)]

    barrier = pltpu.get_barrier_semaphore()
    for dz in range(1, N_Z):
        pl.semaphore_signal(
            barrier, inc=1,
            device_id=(my_x, my_y, (my_z + dz) % N_Z),
            device_id_type=pl.DeviceIdType.MESH,
        )
    pl.semaphore_wait(barrier, N_Z - 1)

    def send(src_ref, dst_ref, t, j):
        rdma = pltpu.make_async_remote_copy(
            src_ref=src_ref,
            dst_ref=dst_ref,
            send_sem=send_sems.at[t, j],
            recv_sem=recv_sems.at[t, my_z],
            device_id=(my_x, my_y, j),
            device_id_type=pl.DeviceIdType.MESH,
        )
        rdma.start()
        return rdma

    rdmas = []
    for dz in range(1, N_Z):
        j = (my_z + dz) % N_Z
        jcol = pl.multiple_of(j * DHB, DHB)
        rdmas.append(send(gc.at[my_z], gc.at[my_z], T_C, j))
        rdmas.append(send(wuk_ref.at[:, pl.ds(jcol, DHB)], guk.at[my_z],
                          T_UK, j))
        rdmas.append(send(wuv_ref.at[:, pl.ds(jcol, DHB)], guv.at[my_z],
                          T_UV, j))

    qs[...] = jnp.dot(xv, wq_ref[:, pl.ds(col0, DHB)],
                      preferred_element_type=f32).astype(bf16)
    qr0 = pl.multiple_of(my_z * DRB, DRB)
    qrs[...] = jnp.dot(xv, wqr_ref[:, pl.ds(qr0, DRB)],
                       preferred_element_type=f32).astype(bf16)
    kr = jnp.dot(xv, wkr_ref[...], preferred_element_type=f32).astype(bf16)

    for t in (T_C, T_UK, T_UV):
        for dz in range(1, N_Z):
            j = (my_z + dz) % N_Z
            pltpu.make_async_remote_copy(
                src_ref=gc.at[my_z], dst_ref=(gc, guk, guv)[t].at[j],
                send_sem=send_sems.at[t, j], recv_sem=recv_sems.at[t, j],
                device_id=(my_x, my_y, j),
                device_id_type=pl.DeviceIdType.MESH,
            ).wait_recv()

    k_acc = jnp.dot(gc[0], guk[0], preferred_element_type=f32)
    for z in range(1, N_Z):
        k_acc = k_acc + jnp.dot(gc[z], guk[z], preferred_element_type=f32)
    ks[...] = k_acc.astype(bf16)
    v_acc = jnp.dot(gc[0], guv[0], preferred_element_type=f32)
    for z in range(1, N_Z):
        v_acc = v_acc + jnp.dot(gc[z], guv[z], preferred_element_type=f32)
    vs[...] = v_acc.astype(bf16)

    for i in range(HB):
        qh = qs[:, i * DH:(i + 1) * DH]
        qrh = qrs[:, i * DR:(i + 1) * DR]
        s = lax.dot_general(qh, ks[:, i * DH:(i + 1) * DH],
                            (((1,), (1,)), ((), ())),
                            preferred_element_type=f32)
        s = s + lax.dot_general(qrh, kr, (((1,), (1,)), ((), ())),
                                preferred_element_type=f32)
        e = jnp.exp(s * SCALE)
        denom = jnp.sum(e, axis=-1, keepdims=True)
        o_un = jnp.dot(e.astype(bf16), vs[:, i * DH:(i + 1) * DH],
                       preferred_element_type=f32)
        o_slots[my_z, i] = (o_un / denom).astype(bf16)
        for dz in range(1, N_Z):
            j = (my_z + dz) % N_Z
            rdma = pltpu.make_async_remote_copy(
                src_ref=o_slots.at[my_z, i], dst_ref=o_slots.at[my_z, i],
                send_sem=o_send_sems.at[j, i],
                recv_sem=o_recv_sems.at[my_z, i],
                device_id=(my_x, my_y, j),
                device_id_type=pl.DeviceIdType.MESH,
            )
            rdma.start()
            rdmas.append(rdma)

    acc = jnp.dot(o_slots[my_z, 0], wo_ref[pl.ds(my_z * DHB, DH), :],
                  preferred_element_type=f32)
    for i in range(1, HB):
        h0 = pl.multiple_of(my_z * DHB + i * DH, DH)
        acc = acc + jnp.dot(o_slots[my_z, i], wo_ref[pl.ds(h0, DH), :],
                            preferred_element_type=f32)
    out_ref[...] = acc
    for dz in range(1, N_Z):
        j = (my_z + dz) % N_Z
        for i in range(HB):
            pltpu.make_async_remote_copy(
                src_ref=o_slots.at[my_z, i], dst_ref=o_slots.at[j, i],
                send_sem=o_send_sems.at[j, i],
                recv_sem=o_recv_sems.at[j, i],
                device_id=(my_x, my_y, j),
                device_id_type=pl.DeviceIdType.MESH,
            ).wait_recv()
        jacc = jnp.dot(o_slots[j, 0], wo_ref[pl.ds(j * DHB, DH), :],
                       preferred_element_type=f32)
        for i in range(1, HB):
            h0 = pl.multiple_of(j * DHB + i * DH, DH)
            jacc = jacc + jnp.dot(o_slots[j, i], wo_ref[pl.ds(h0, DH), :],
                                  preferred_element_type=f32)
        out_ref[...] = out_ref[...] + jacc

    for rdma in rdmas:
        rdma.wait_send()


def kernel(x, Wdkv, Wuk, Wuv, Wq, Wqr, Wkr, Wo):
    bf16 = jnp.bfloat16
    xb = x[0].astype(bf16)

    out = pl.pallas_call(
        _body,
        out_shape=jax.ShapeDtypeStruct((S, D), jnp.float32),
        in_specs=[pl.BlockSpec(memory_space=pltpu.VMEM)] * 8,
        out_specs=pl.BlockSpec(memory_space=pltpu.VMEM),
        scratch_shapes=[
            pltpu.VMEM((N_Z, S, DC), bf16),
            pltpu.VMEM((N_Z, DC, DHB), bf16),
            pltpu.VMEM((N_Z, DC, DHB), bf16),
            pltpu.VMEM((S, DHB), bf16),
            pltpu.VMEM((S, DRB), bf16),
            pltpu.VMEM((S, DHB), bf16),
            pltpu.VMEM((S, DHB), bf16),
            pltpu.VMEM((N_Z, HB, S, DH), bf16),
            pltpu.SemaphoreType.DMA((4, N_Z)),
            pltpu.SemaphoreType.DMA((4, N_Z)),
            pltpu.SemaphoreType.DMA((N_Z, HB)),
            pltpu.SemaphoreType.DMA((N_Z, HB)),
        ],
        compiler_params=pltpu.CompilerParams(
            collective_id=0, vmem_limit_bytes=128 * 1024 * 1024),
    )(xb, Wdkv.astype(bf16), Wuk.astype(bf16), Wuv.astype(bf16),
      Wq.astype(bf16), Wqr.astype(bf16), Wkr.astype(bf16), Wo.astype(bf16))
    return out.reshape(1, S, D)


# device time: 110787 ns/iter; 1.2085x vs baseline; 1.2085x over previous
import jax
import jax.numpy as jnp
from jax import lax
from jax.experimental import pallas as pl
from jax.experimental.pallas import tpu as pltpu

N_Z = 4
S = 1024
D = 2048
DC = 128
H = 16
DH = 128
DR = 32
HB = H // N_Z
DHB = HB * DH
DRB = HB * DR
SCALE = (DH + DR) ** -0.5

T_C, T_UK, T_UV, T_O = 0, 1, 2, 3


def _body(x_ref, wdkv_ref, wuk_ref, wuv_ref, wq_hbm, wqr_hbm, wkr_ref,
          wo_ref, out_ref, gc, guk, guv, wqf, wqrf, qs, qrs, ks, vs,
          o_slots, send_sems, recv_sems, copy_sems):
    f32 = jnp.float32
    bf16 = jnp.bfloat16
    my_x = lax.axis_index("x")
    my_y = lax.axis_index("y")
    my_z = lax.axis_index("z")
    col0 = pl.multiple_of(my_z * DHB, DHB)
    qr0 = pl.multiple_of(my_z * DRB, DRB)

    wq_cp = pltpu.make_async_copy(
        wq_hbm.at[:, pl.ds(col0, DHB)], wqf, copy_sems.at[0])
    wq_cp.start()
    wqr_cp = pltpu.make_async_copy(
        wqr_hbm.at[:, pl.ds(qr0, DRB)], wqrf, copy_sems.at[1])
    wqr_cp.start()

    xv = x_ref[...]

    c_loc = jnp.dot(xv, wdkv_ref[...], preferred_element_type=f32)
    gc[my_z] = c_loc.astype(bf16)
    guk[my_z] = wuk_ref[:, pl.ds(col0, DHB)]
    guv[my_z] = wuv_ref[:, pl.ds(col0, DHB)]

    barrier = pltpu.get_barrier_semaphore()
    for dz in range(1, N_Z):
        pl.semaphore_signal(
            barrier, inc=1,
            device_id=(my_x, my_y, (my_z + dz) % N_Z),
            device_id_type=pl.DeviceIdType.MESH,
        )
    pl.semaphore_wait(barrier, N_Z - 1)

    def send(src_ref, dst_ref, t, j):
        rdma = pltpu.make_async_remote_copy(
            src_ref=src_ref,
            dst_ref=dst_ref,
            send_sem=send_sems.at[t, j],
            recv_sem=recv_sems.at[t, my_z],
            device_id=(my_x, my_y, j),
            device_id_type=pl.DeviceIdType.MESH,
        )
        rdma.start()
        return rdma

    rdmas = []
    for dz in range(1, N_Z):
        j = (my_z + dz) % N_Z
        jcol = pl.multiple_of(j * DHB, DHB)
        rdmas.append(send(gc.at[my_z], gc.at[my_z], T_C, j))
        rdmas.append(send(wuk_ref.at[:, pl.ds(jcol, DHB)], guk.at[my_z],
                          T_UK, j))
        rdmas.append(send(wuv_ref.at[:, pl.ds(jcol, DHB)], guv.at[my_z],
                          T_UV, j))

    wq_cp.wait()
    qs[...] = jnp.dot(xv, wqf[...].astype(bf16),
                      preferred_element_type=f32).astype(bf16)
    wqr_cp.wait()
    qrs[...] = jnp.dot(xv, wqrf[...].astype(bf16),
                       preferred_element_type=f32).astype(bf16)
    kr = jnp.dot(xv, wkr_ref[...], preferred_element_type=f32).astype(bf16)

    for t in (T_C, T_UK, T_UV):
        for dz in range(1, N_Z):
            j = (my_z + dz) % N_Z
            pltpu.make_async_remote_copy(
                src_ref=gc.at[my_z], dst_ref=(gc, guk, guv)[t].at[j],
                send_sem=send_sems.at[t, j], recv_sem=recv_sems.at[t, j],
                device_id=(my_x, my_y, j),
                device_id_type=pl.DeviceIdType.MESH,
            ).wait_recv()

    k_acc = jnp.dot(gc[0], guk[0], preferred_element_type=f32)
    for z in range(1, N_Z):
        k_acc = k_acc + jnp.dot(gc[z], guk[z], preferred_element_type=f32)
    ks[...] = k_acc.astype(bf16)
    v_acc = jnp.dot(gc[0], guv[0], preferred_element_type=f32)
    for z in range(1, N_Z):
        v_acc = v_acc + jnp.dot(gc[z], guv[z], preferred_element_type=f32)
    vs[...] = v_acc.astype(bf16)

    for i in range(HB):
        qh = qs[:, i * DH:(i + 1) * DH]
        qrh = qrs[:, i * DR:(i + 1) * DR]
        s = lax.dot_general(qh, ks[:, i * DH:(i + 1) * DH],
                            (((1,), (1,)), ((), ())),
                            preferred_element_type=f32)
        s = s + lax.dot_general(qrh, kr, (((1,), (1,)), ((), ())),
                                preferred_element_type=f32)
        e = jnp.exp(s * SCALE)
        denom = jnp.sum(e, axis=-1, keepdims=True)
        o_un = jnp.dot(e.astype(bf16), vs[:, i * DH:(i + 1) * DH],
                       preferred_element_type=f32)
        o_slots[my_z, :, i * DH:(i + 1) * DH] = (o_un / denom).astype(bf16)

    for dz in range(1, N_Z):
        j = (my_z + dz) % N_Z
        rdmas.append(send(o_slots.at[my_z], o_slots.at[my_z], T_O, j))

    row0 = pl.multiple_of(my_z * DHB, DHB)
    out_ref[...] = jnp.dot(o_slots[my_z], wo_ref[pl.ds(row0, DHB), :],
                           preferred_element_type=f32)
    for dz in range(1, N_Z):
        j = (my_z + dz) % N_Z
        pltpu.make_async_remote_copy(
            src_ref=o_slots.at[my_z], dst_ref=o_slots.at[j],
            send_sem=send_sems.at[T_O, j], recv_sem=recv_sems.at[T_O, j],
            device_id=(my_x, my_y, j),
            device_id_type=pl.DeviceIdType.MESH,
        ).wait_recv()
        jrow = pl.multiple_of(j * DHB, DHB)
        out_ref[...] = out_ref[...] + jnp.dot(
            o_slots[j], wo_ref[pl.ds(jrow, DHB), :],
            preferred_element_type=f32)

    for rdma in rdmas:
        rdma.wait_send()


def kernel(x, Wdkv, Wuk, Wuv, Wq, Wqr, Wkr, Wo):
    bf16 = jnp.bfloat16
    xb = x[0].astype(bf16)

    out = pl.pallas_call(
        _body,
        out_shape=jax.ShapeDtypeStruct((S, D), jnp.float32),
        in_specs=[
            pl.BlockSpec(memory_space=pltpu.VMEM),
            pl.BlockSpec(memory_space=pltpu.VMEM),
            pl.BlockSpec(memory_space=pltpu.VMEM),
            pl.BlockSpec(memory_space=pltpu.VMEM),
            pl.BlockSpec(memory_space=pl.ANY),
            pl.BlockSpec(memory_space=pl.ANY),
            pl.BlockSpec(memory_space=pltpu.VMEM),
            pl.BlockSpec(memory_space=pltpu.VMEM),
        ],
        out_specs=pl.BlockSpec(memory_space=pltpu.VMEM),
        scratch_shapes=[
            pltpu.VMEM((N_Z, S, DC), bf16),
            pltpu.VMEM((N_Z, DC, DHB), bf16),
            pltpu.VMEM((N_Z, DC, DHB), bf16),
            pltpu.VMEM((D, DHB), jnp.float32),
            pltpu.VMEM((D, DRB), jnp.float32),
            pltpu.VMEM((S, DHB), bf16),
            pltpu.VMEM((S, DRB), bf16),
            pltpu.VMEM((S, DHB), bf16),
            pltpu.VMEM((S, DHB), bf16),
            pltpu.VMEM((N_Z, S, DHB), bf16),
            pltpu.SemaphoreType.DMA((4, N_Z)),
            pltpu.SemaphoreType.DMA((4, N_Z)),
            pltpu.SemaphoreType.DMA((2,)),
        ],
        compiler_params=pltpu.CompilerParams(
            collective_id=0, vmem_limit_bytes=128 * 1024 * 1024),
    )(xb, Wdkv.astype(bf16), Wuk.astype(bf16), Wuv.astype(bf16),
      Wq, Wqr, Wkr.astype(bf16), Wo.astype(bf16))
    return out.reshape(1, S, D)
